# baseline (speedup 1.0000x reference)
import jax
import jax.numpy as jnp
from jax.experimental import pallas as pl
from jax.experimental.pallas import tpu as pltpu


def _support_kernel(x_ref, w1_ref, o_ref):
    o_ref[...] = jnp.dot(x_ref[...], w1_ref[...],
                         preferred_element_type=jnp.float32)


def _layer1_kernel(adj_ref, sup_ref, b1_ref, w2_ref, o_ref):
    h = jnp.dot(adj_ref[...], sup_ref[...],
                preferred_element_type=jnp.float32)
    h = jnp.maximum(h + b1_ref[...], 0.0)
    o_ref[...] = jnp.dot(h, w2_ref[...],
                         preferred_element_type=jnp.float32)


def _layer2_kernel(adj_ref, s2_ref, b2_ref, o_ref):
    logits = jnp.dot(adj_ref[...], s2_ref[...],
                     preferred_element_type=jnp.float32) + b2_ref[...]
    m = jnp.max(logits, axis=1, keepdims=True)
    z = logits - m
    lse = jnp.log(jnp.sum(jnp.exp(z), axis=1, keepdims=True))
    o_ref[...] = z - lse


def kernel(x, adj, W1, b1, W2, b2):
    n, _ = x.shape
    h_dim = W1.shape[1]
    c_dim = W2.shape[1]
    bm = 80

    b1_2d = b1.reshape(1, h_dim)
    b2_2d = b2.reshape(1, c_dim)

    support = pl.pallas_call(
        _support_kernel,
        out_shape=jax.ShapeDtypeStruct((n, h_dim), jnp.float32),
    )(x, W1)

    grid = (n // bm,)
    params = pltpu.CompilerParams(dimension_semantics=("parallel",))

    s2 = pl.pallas_call(
        _layer1_kernel,
        grid=grid,
        in_specs=[
            pl.BlockSpec((bm, n), lambda i: (i, 0)),
            pl.BlockSpec((n, h_dim), lambda i: (0, 0)),
            pl.BlockSpec((1, h_dim), lambda i: (0, 0)),
            pl.BlockSpec((h_dim, c_dim), lambda i: (0, 0)),
        ],
        out_specs=pl.BlockSpec((bm, c_dim), lambda i: (i, 0)),
        out_shape=jax.ShapeDtypeStruct((n, c_dim), jnp.float32),
        compiler_params=params,
    )(adj, support, b1_2d, W2)

    out = pl.pallas_call(
        _layer2_kernel,
        grid=grid,
        in_specs=[
            pl.BlockSpec((bm, n), lambda i: (i, 0)),
            pl.BlockSpec((n, c_dim), lambda i: (0, 0)),
            pl.BlockSpec((1, c_dim), lambda i: (0, 0)),
        ],
        out_specs=pl.BlockSpec((bm, c_dim), lambda i: (i, 0)),
        out_shape=jax.ShapeDtypeStruct((n, c_dim), jnp.float32),
        compiler_params=params,
    )(adj, s2, b2_2d)

    return out


# speedup vs baseline: 1.4737x; 1.4737x over previous
import jax
import jax.numpy as jnp
from jax.experimental import pallas as pl
from jax.experimental.pallas import tpu as pltpu


def _gcn_kernel(x_ref, w1_ref, b1_ref, w2_ref, b2_ref, adj_ref, o_ref,
                sup_s, s2_s, *, bm):
    p = pl.program_id(0)
    i = pl.program_id(1)

    @pl.when((p == 0) & (i == 0))
    def _():
        sup_s[...] = jnp.dot(x_ref[...], w1_ref[...],
                             preferred_element_type=jnp.float32)

    @pl.when(p == 0)
    def _():
        h = jnp.dot(adj_ref[...], sup_s[...],
                    preferred_element_type=jnp.float32)
        h = jnp.maximum(h + b1_ref[...], 0.0)
        s2 = jnp.dot(h, w2_ref[...], preferred_element_type=jnp.float32)
        s2_s[pl.ds(i * bm, bm), :] = s2
        o_ref[...] = s2

    @pl.when(p == 1)
    def _():
        logits = jnp.dot(adj_ref[...], s2_s[...],
                         preferred_element_type=jnp.float32) + b2_ref[...]
        m = jnp.max(logits, axis=1, keepdims=True)
        z = logits - m
        lse = jnp.log(jnp.sum(jnp.exp(z), axis=1, keepdims=True))
        o_ref[...] = z - lse


def kernel(x, adj, W1, b1, W2, b2):
    n, f_in = x.shape
    h_dim = W1.shape[1]
    c_dim = W2.shape[1]
    bm = 400
    nb = n // bm

    import functools
    body = functools.partial(_gcn_kernel, bm=bm)

    return pl.pallas_call(
        body,
        grid=(2, nb),
        in_specs=[
            pl.BlockSpec((n, f_in), lambda p, i: (0, 0)),
            pl.BlockSpec((f_in, h_dim), lambda p, i: (0, 0)),
            pl.BlockSpec((1, h_dim), lambda p, i: (0, 0)),
            pl.BlockSpec((h_dim, c_dim), lambda p, i: (0, 0)),
            pl.BlockSpec((1, c_dim), lambda p, i: (0, 0)),
            pl.BlockSpec((bm, n), lambda p, i: (i, 0)),
        ],
        out_specs=pl.BlockSpec((bm, c_dim), lambda p, i: (i, 0)),
        out_shape=jax.ShapeDtypeStruct((n, c_dim), jnp.float32),
        scratch_shapes=[
            pltpu.VMEM((n, h_dim), jnp.float32),
            pltpu.VMEM((n, c_dim), jnp.float32),
        ],
        compiler_params=pltpu.CompilerParams(
            dimension_semantics=("arbitrary", "arbitrary")),
    )(x, W1, b1.reshape(1, h_dim), W2, b2.reshape(1, c_dim), adj)


# speedup vs baseline: 1.4753x; 1.0011x over previous
import jax
import jax.numpy as jnp
from jax.experimental import pallas as pl
from jax.experimental.pallas import tpu as pltpu


def _gcn_kernel(x_ref, w1_ref, b1_ref, w2_ref, b2_ref, adj_ref, o_ref,
                sup_s, s2_s, *, bm):
    p = pl.program_id(0)
    i = pl.program_id(1)

    @pl.when((p == 0) & (i == 0))
    def _():
        sup_s[...] = jnp.dot(x_ref[...], w1_ref[...],
                             preferred_element_type=jnp.float32)

    @pl.when(p == 0)
    def _():
        h = jnp.dot(adj_ref[...], sup_s[...],
                    preferred_element_type=jnp.float32)
        h = jnp.maximum(h + b1_ref[...], 0.0)
        s2 = jnp.dot(h, w2_ref[...], preferred_element_type=jnp.float32)
        s2_s[pl.ds(i * bm, bm), :] = s2
        o_ref[...] = s2

    @pl.when(p == 1)
    def _():
        logits = jnp.dot(adj_ref[...], s2_s[...],
                         preferred_element_type=jnp.float32) + b2_ref[...]
        m = jnp.max(logits, axis=1, keepdims=True)
        z = logits - m
        lse = jnp.log(jnp.sum(jnp.exp(z), axis=1, keepdims=True))
        o_ref[...] = z - lse


def kernel(x, adj, W1, b1, W2, b2):
    n, f_in = x.shape
    h_dim = W1.shape[1]
    c_dim = W2.shape[1]
    bm = 400
    nb = n // bm

    import functools
    body = functools.partial(_gcn_kernel, bm=bm)

    def row_map(p, i):
        return (p * (nb - 1) + (1 - 2 * p) * i, 0)

    return pl.pallas_call(
        body,
        grid=(2, nb),
        in_specs=[
            pl.BlockSpec((n, f_in), lambda p, i: (0, 0)),
            pl.BlockSpec((f_in, h_dim), lambda p, i: (0, 0)),
            pl.BlockSpec((1, h_dim), lambda p, i: (0, 0)),
            pl.BlockSpec((h_dim, c_dim), lambda p, i: (0, 0)),
            pl.BlockSpec((1, c_dim), lambda p, i: (0, 0)),
            pl.BlockSpec((bm, n), row_map),
        ],
        out_specs=pl.BlockSpec((bm, c_dim), row_map),
        out_shape=jax.ShapeDtypeStruct((n, c_dim), jnp.float32),
        scratch_shapes=[
            pltpu.VMEM((n, h_dim), jnp.float32),
            pltpu.VMEM((n, c_dim), jnp.float32),
        ],
        compiler_params=pltpu.CompilerParams(
            dimension_semantics=("arbitrary", "arbitrary")),
    )(x, W1, b1.reshape(1, h_dim), W2, b2.reshape(1, c_dim), adj)
